# baseline (speedup 1.0000x reference)
import functools

import jax
import jax.numpy as jnp
from jax.experimental import pallas as pl
from jax.experimental.pallas import tpu as pltpu

_TEMP = 0.05
_NB = 2048


def _logits_body(x_ref, f_ref, o_ref):
    x = x_ref[...]
    norm = jnp.sqrt(jnp.sum(x * x, axis=1, keepdims=True))
    xs = x * ((1.0 / _TEMP) / jnp.maximum(norm, 1e-12))
    o_ref[...] = jax.lax.dot_general(
        xs,
        f_ref[...],
        (((1,), (1,)), ((), ())),
        preferred_element_type=jnp.float32,
    )


@functools.partial(jax.jit, static_argnames=())
def kernel(inputs, targets, features):
    del targets
    b, d = inputs.shape
    n = features.shape[0]
    grid = (pl.cdiv(n, _NB),)
    return pl.pallas_call(
        _logits_body,
        grid=grid,
        in_specs=[
            pl.BlockSpec((b, d), lambda i: (0, 0)),
            pl.BlockSpec((_NB, d), lambda i: (i, 0)),
        ],
        out_specs=pl.BlockSpec((b, _NB), lambda i: (0, i)),
        out_shape=jax.ShapeDtypeStruct((b, n), jnp.float32),
        compiler_params=pltpu.CompilerParams(
            dimension_semantics=("parallel",),
        ),
    )(inputs, features)


# speedup vs baseline: 1.0007x; 1.0007x over previous
import functools

import jax
import jax.numpy as jnp
from jax.experimental import pallas as pl
from jax.experimental.pallas import tpu as pltpu
from jax.experimental.layout import Format, Layout, with_layout_constraint

_TEMP = 0.05
_NB = 2048


def _logits_body(x_ref, f_ref, o_ref):
    x = x_ref[...]
    norm = jnp.sqrt(jnp.sum(x * x, axis=1, keepdims=True))
    xs = x * ((1.0 / _TEMP) / jnp.maximum(norm, 1e-12))
    o_ref[...] = jax.lax.dot_general(
        xs,
        f_ref[...],
        (((1,), (1,)), ((), ())),
        preferred_element_type=jnp.float32,
    )


@functools.partial(jax.jit, static_argnames=())
def kernel(inputs, targets, features):
    del targets
    b, d = inputs.shape
    n = features.shape[0]
    grid = (pl.cdiv(n, _NB),)
    out = pl.pallas_call(
        _logits_body,
        grid=grid,
        in_specs=[
            pl.BlockSpec((b, d), lambda i: (0, 0)),
            pl.BlockSpec((_NB, d), lambda i: (i, 0)),
        ],
        out_specs=pl.BlockSpec((b, _NB), lambda i: (0, i)),
        out_shape=jax.ShapeDtypeStruct((b, n), jnp.float32),
        compiler_params=pltpu.CompilerParams(
            dimension_semantics=("parallel",),
        ),
    )(inputs, features)
    return with_layout_constraint(
        out, Layout(major_to_minor=(0, 1), tiling=((8, 128),))
    )


# speedup vs baseline: 4.0114x; 4.0087x over previous
import jax
import jax.numpy as jnp
from jax.experimental import pallas as pl
from jax.experimental.pallas import tpu as pltpu

_TEMP = 0.05
_NB = 2048


def _logits_t_body(xt_ref, ft_ref, o_ref):
    xt = xt_ref[...]
    norm = jnp.sqrt(jnp.sum(xt * xt, axis=0, keepdims=True))
    xs = xt * ((1.0 / _TEMP) / jnp.maximum(norm, 1e-12))
    o_ref[...] = jax.lax.dot_general(
        ft_ref[...],
        xs,
        (((0,), (0,)), ((), ())),
        preferred_element_type=jnp.float32,
    )


def kernel(inputs, targets, features):
    del targets
    b, d = inputs.shape
    n = features.shape[0]
    xt = jnp.transpose(inputs)
    ft = jnp.transpose(features)
    out_t = pl.pallas_call(
        _logits_t_body,
        grid=(pl.cdiv(n, _NB),),
        in_specs=[
            pl.BlockSpec((d, b), lambda i: (0, 0)),
            pl.BlockSpec((d, _NB), lambda i: (0, i)),
        ],
        out_specs=pl.BlockSpec((_NB, b), lambda i: (i, 0)),
        out_shape=jax.ShapeDtypeStruct((n, b), jnp.float32),
        compiler_params=pltpu.CompilerParams(
            dimension_semantics=("arbitrary",),
        ),
    )(xt, ft)
    return jnp.transpose(out_t)


# speedup vs baseline: 4.0628x; 1.0128x over previous
import jax
import jax.numpy as jnp
from jax.experimental import pallas as pl
from jax.experimental.pallas import tpu as pltpu

_TEMP = 0.05
_NB = 4096


def _logits_t_body(xt_ref, ft_ref, o_ref):
    xt = xt_ref[...]
    norm = jnp.sqrt(jnp.sum(xt * xt, axis=0, keepdims=True))
    xs = xt * ((1.0 / _TEMP) / jnp.maximum(norm, 1e-12))
    o_ref[...] = jax.lax.dot_general(
        ft_ref[...],
        xs,
        (((0,), (0,)), ((), ())),
        preferred_element_type=jnp.float32,
    )


def kernel(inputs, targets, features):
    del targets
    b, d = inputs.shape
    n = features.shape[0]
    xt = jnp.transpose(inputs)
    ft = jnp.transpose(features)
    out_t = pl.pallas_call(
        _logits_t_body,
        grid=(pl.cdiv(n, _NB),),
        in_specs=[
            pl.BlockSpec((d, b), lambda i: (0, 0)),
            pl.BlockSpec((d, _NB), lambda i: (0, i)),
        ],
        out_specs=pl.BlockSpec((_NB, b), lambda i: (i, 0)),
        out_shape=jax.ShapeDtypeStruct((n, b), jnp.float32),
        compiler_params=pltpu.CompilerParams(
            dimension_semantics=("arbitrary",),
        ),
    )(xt, ft)
    return jnp.transpose(out_t)
